# baseline (speedup 1.0000x reference)
import functools

import jax
import jax.numpy as jnp
from jax import lax
from jax.experimental import pallas as pl
from jax.experimental.pallas import tpu as pltpu
from jax.experimental.pallas import tpu_sc as plsc

NC = 2
NS = 16
NW = NC * NS


def _gather_body(bpw, L, table_hbm, words_hbm, out_hbm,
                 idx_v, gb_a, gb_b, sg_a, sg_b, sw_a, sw_b):
    wid = lax.axis_index("s") * NC + lax.axis_index("c")
    b0 = wid * bpw
    n_pairs = bpw // 2

    pltpu.sync_copy(words_hbm.at[pl.ds(b0, bpw)], idx_v)

    def start_gather(b, gb, sg):
        pltpu.async_copy(table_hbm.at[idx_v.at[b]], gb, sg)

    def wait_gather(b, gb, sg):
        pltpu.make_async_copy(table_hbm.at[idx_v.at[b]], gb, sg).wait()

    def start_wb(b, gb, sw):
        pltpu.async_copy(gb, out_hbm.at[b0 + b], sw)

    def wait_wb(gb, sw):
        pltpu.make_async_copy(gb, out_hbm.at[b0], sw).wait()

    start_gather(0, gb_a, sg_a)

    def pair(p, carry):
        e = p * 2
        o = e + 1

        @pl.when(p >= 1)
        def _():
            wait_wb(gb_b, sw_b)

        start_gather(o, gb_b, sg_b)
        wait_gather(e, gb_a, sg_a)
        start_wb(e, gb_a, sw_a)

        wait_wb(gb_a, sw_a)

        @pl.when(p + 1 < n_pairs)
        def _():
            start_gather(e + 2, gb_a, sg_a)

        wait_gather(o, gb_b, sg_b)
        start_wb(o, gb_b, sw_b)
        return carry

    lax.fori_loop(0, n_pairs, pair, 0)
    wait_wb(gb_b, sw_b)


@jax.jit
def kernel(words, word_emb):
    B, L = words.shape
    V, D = word_emb.shape
    if words.dtype != jnp.int32:
        words = words.astype(jnp.int32)

    bpw = B // NW
    mesh = plsc.VectorSubcoreMesh(core_axis_name="c", subcore_axis_name="s")
    body = functools.partial(_gather_body, bpw, L)
    out = pl.kernel(
        body,
        out_type=jax.ShapeDtypeStruct((B, L, D), jnp.float32),
        mesh=mesh,
        compiler_params=pltpu.CompilerParams(use_tc_tiling_on_sc=False),
        scratch_types=[
            pltpu.VMEM((bpw, L), jnp.int32),
            pltpu.VMEM((L, D), jnp.float32),
            pltpu.VMEM((L, D), jnp.float32),
            pltpu.SemaphoreType.DMA,
            pltpu.SemaphoreType.DMA,
            pltpu.SemaphoreType.DMA,
            pltpu.SemaphoreType.DMA,
        ],
    )(word_emb, words)
    return out
